# Initial kernel scaffold; baseline (speedup 1.0000x reference)
import jax
import jax.numpy as jnp
from jax.experimental import pallas as pl


def kernel(x, edge_index, edge_weight, batch, smri_data, W1, b1, W2, b2, Ws, bs, Wf, bf):
    raise NotImplementedError("write your pallas kernel here")


# speedup vs baseline: 10.2795x; 10.2795x over previous
import functools

import jax
import jax.numpy as jnp
from jax import lax
from jax.experimental import pallas as pl
from jax.experimental.pallas import tpu as pltpu
from jax.experimental.pallas import tpu_sc as plsc

N_NODES = 10000
N_EDGES = 320000
NFEAT = 128
NGRAPH = 16

NC = 2
NS = 16
NW = NC * NS
CHUNK = 128
NCHUNKS = N_EDGES // CHUNK
NPAD = 10240
ROWS_PER_TILE = NPAD // NS
ZROWS = 128
DEGW = 16


def _worker_id():
  return lax.axis_index("s") * NC + lax.axis_index("c")


def _num_chunks(wid):
  base = NCHUNKS // NW
  extra = NCHUNKS - base * NW
  return jnp.where(wid < extra, base + 1, base)


def _sc_deg_body(ew_hbm, col_hbm, out_hbm, ewv, colv, valbuf, zbuf, acc):
  wid = _worker_id()
  sid = lax.axis_index("s")
  cid = lax.axis_index("c")
  zeros16 = jnp.zeros((16,), jnp.float32)

  def zzrow(i, _):
    zbuf[i, :] = zeros16
    return 0
  lax.fori_loop(0, ZROWS, zzrow, 0)
  for r in range(ROWS_PER_TILE // ZROWS):
    pltpu.sync_copy(zbuf, acc.at[pl.ds(sid * ROWS_PER_TILE + r * ZROWS, ZROWS)])
  plsc.subcore_barrier()

  def body(k, _):
    ck = wid + NW * k
    pltpu.sync_copy(ew_hbm.at[ck], ewv)
    pltpu.sync_copy(col_hbm.at[ck], colv)

    def fill(i, _):
      valbuf[i, :] = plsc.load_gather(ewv, [jnp.full((16,), i, jnp.int32)])
      return 0

    lax.fori_loop(0, CHUNK, fill, 0)
    pltpu.sync_copy(valbuf, acc.at[colv], add=True)
    return 0

  lax.fori_loop(0, _num_chunks(wid), body, 0)
  plsc.subcore_barrier()
  pltpu.sync_copy(acc.at[pl.ds(sid * ROWS_PER_TILE, ROWS_PER_TILE)],
                  out_hbm.at[cid, pl.ds(sid * ROWS_PER_TILE, ROWS_PER_TILE)])


def _sc_deg(ew2d, col2d):
  mesh = plsc.VectorSubcoreMesh(core_axis_name="c", subcore_axis_name="s")
  return pl.kernel(
      _sc_deg_body,
      out_type=jax.ShapeDtypeStruct((NC, NPAD, DEGW), jnp.float32),
      mesh=mesh,
      compiler_params=pltpu.CompilerParams(needs_layout_passes=False),
      scratch_types=[
          pltpu.VMEM((CHUNK,), jnp.float32),
          pltpu.VMEM((CHUNK,), jnp.int32),
          pltpu.VMEM((CHUNK, DEGW), jnp.float32),
          pltpu.VMEM((ZROWS, DEGW), jnp.float32),
          pltpu.VMEM_SHARED((NPAD, DEGW), jnp.float32),
      ],
  )(ew2d, col2d)


def _sc_msg_body(xn_hbm, row_hbm, col_hbm, ew_hbm, out_hbm,
                 rowv, colv, ewv, rows, zbuf, sem, acc):
  wid = _worker_id()
  sid = lax.axis_index("s")
  cid = lax.axis_index("c")
  zeros16 = jnp.zeros((16,), jnp.float32)

  def zzrow(i, _):
    for j in range(NFEAT // 16):
      zbuf[i, pl.ds(j * 16, 16)] = zeros16
    return 0
  lax.fori_loop(0, ZROWS, zzrow, 0)
  for r in range(ROWS_PER_TILE // ZROWS):
    pltpu.sync_copy(zbuf, acc.at[pl.ds(sid * ROWS_PER_TILE + r * ZROWS, ZROWS)])
  plsc.subcore_barrier()

  def body(k, _):
    ck = wid + NW * k
    pltpu.sync_copy(row_hbm.at[ck], rowv)
    pltpu.sync_copy(col_hbm.at[ck], colv)
    pltpu.sync_copy(ew_hbm.at[ck], ewv)
    pltpu.async_copy(xn_hbm.at[rowv], rows, sem).wait()

    def scale(i, _):
      coef = plsc.load_gather(ewv, [jnp.full((16,), i, jnp.int32)])
      for j in range(NFEAT // 16):
        sl = pl.ds(j * 16, 16)
        rows[i, sl] = rows[i, sl] * coef
      return 0

    lax.fori_loop(0, CHUNK, scale, 0)
    pltpu.sync_copy(rows, acc.at[colv], add=True)
    return 0

  lax.fori_loop(0, _num_chunks(wid), body, 0)
  plsc.subcore_barrier()
  pltpu.sync_copy(acc.at[pl.ds(sid * ROWS_PER_TILE, ROWS_PER_TILE)],
                  out_hbm.at[cid, pl.ds(sid * ROWS_PER_TILE, ROWS_PER_TILE)])


def _sc_msg(xn, row2d, col2d, ew2d):
  mesh = plsc.VectorSubcoreMesh(core_axis_name="c", subcore_axis_name="s")
  return pl.kernel(
      _sc_msg_body,
      out_type=jax.ShapeDtypeStruct((NC, NPAD, NFEAT), jnp.float32),
      mesh=mesh,
      compiler_params=pltpu.CompilerParams(needs_layout_passes=False),
      scratch_types=[
          pltpu.VMEM((CHUNK,), jnp.int32),
          pltpu.VMEM((CHUNK,), jnp.int32),
          pltpu.VMEM((CHUNK,), jnp.float32),
          pltpu.VMEM((CHUNK, NFEAT), jnp.float32),
          pltpu.VMEM((ZROWS, NFEAT), jnp.float32),
          pltpu.SemaphoreType.DMA,
          pltpu.VMEM_SHARED((NPAD, NFEAT), jnp.float32),
      ],
  )(xn, row2d, col2d, ew2d)


def _tc_a_body(x_ref, w1_ref, degp_ref, xw_ref, xn_ref, dis_ref):
  xw = jnp.dot(x_ref[...], w1_ref[...], preferred_element_type=jnp.float32)
  deg = degp_ref[0, :, 0:1] + degp_ref[1, :, 0:1] + 1.0
  dis = lax.rsqrt(deg)
  xw_ref[...] = xw
  xn_ref[...] = xw * dis
  dis_ref[...] = dis


def _tc_a(x, W1, degp):
  nb = 10
  blk = N_NODES // nb
  return pl.pallas_call(
      _tc_a_body,
      grid=(nb,),
      in_specs=[
          pl.BlockSpec((blk, NFEAT), lambda i: (i, 0)),
          pl.BlockSpec((NFEAT, NFEAT), lambda i: (0, 0)),
          pl.BlockSpec((NC, blk, DEGW), lambda i: (0, i, 0)),
      ],
      out_specs=[
          pl.BlockSpec((blk, NFEAT), lambda i: (i, 0)),
          pl.BlockSpec((blk, NFEAT), lambda i: (i, 0)),
          pl.BlockSpec((blk, 1), lambda i: (i, 0)),
      ],
      out_shape=[
          jax.ShapeDtypeStruct((N_NODES, NFEAT), jnp.float32),
          jax.ShapeDtypeStruct((N_NODES, NFEAT), jnp.float32),
          jax.ShapeDtypeStruct((N_NODES, 1), jnp.float32),
      ],
  )(x, W1, degp)


def _tc_b_body(accp_ref, xw1_ref, dis_ref, b1_ref, w2_ref, xw2_ref, xn2_ref):
  dis = dis_ref[...]
  acc = accp_ref[0] + accp_ref[1]
  h1 = jax.nn.relu(dis * acc + xw1_ref[...] * dis * dis + b1_ref[...])
  xw2 = jnp.dot(h1, w2_ref[...], preferred_element_type=jnp.float32)
  xw2_ref[...] = xw2
  xn2_ref[...] = xw2 * dis


def _tc_b(accp, xw1, dis, b1r, W2):
  nb = 10
  blk = N_NODES // nb
  return pl.pallas_call(
      _tc_b_body,
      grid=(nb,),
      in_specs=[
          pl.BlockSpec((NC, blk, NFEAT), lambda i: (0, i, 0)),
          pl.BlockSpec((blk, NFEAT), lambda i: (i, 0)),
          pl.BlockSpec((blk, 1), lambda i: (i, 0)),
          pl.BlockSpec((1, NFEAT), lambda i: (0, 0)),
          pl.BlockSpec((NFEAT, NFEAT), lambda i: (0, 0)),
      ],
      out_specs=[
          pl.BlockSpec((blk, NFEAT), lambda i: (i, 0)),
          pl.BlockSpec((blk, NFEAT), lambda i: (i, 0)),
      ],
      out_shape=[
          jax.ShapeDtypeStruct((N_NODES, NFEAT), jnp.float32),
          jax.ShapeDtypeStruct((N_NODES, NFEAT), jnp.float32),
      ],
  )(accp, xw1, dis, b1r, W2)


def _tc_c_body(accp_ref, xw2_ref, dis_ref, b2_ref, batch_ref, smri_ref,
               ws_ref, bs_ref, wf1_ref, wf2_ref, bf_ref, out_ref,
               pooled_ref, counts_ref):
  i = pl.program_id(0)
  nb = pl.num_programs(0)

  @pl.when(i == 0)
  def _():
    pooled_ref[...] = jnp.zeros_like(pooled_ref)
    counts_ref[...] = jnp.zeros_like(counts_ref)

  dis = dis_ref[...]
  acc = accp_ref[0] + accp_ref[1]
  h2 = jax.nn.relu(dis * acc + xw2_ref[...] * dis * dis + b2_ref[...])

  bt = batch_ref[0, 0, :]
  blk = h2.shape[0]
  onehot = (bt[:, None] == lax.broadcasted_iota(jnp.int32, (blk, NGRAPH), 1))
  onehot = onehot.astype(jnp.float32)
  pooled_ref[...] += lax.dot_general(
      onehot, h2, (((0,), (0,)), ((), ())), preferred_element_type=jnp.float32)
  ones8 = jnp.ones((blk, 8), jnp.float32)
  counts_ref[...] += lax.dot_general(
      onehot, ones8, (((0,), (0,)), ((), ())), preferred_element_type=jnp.float32)

  @pl.when(i == nb - 1)
  def _():
    cnt = jnp.maximum(counts_ref[:, 0:1], 1.0)
    mean = pooled_ref[...] / cnt
    semb = jax.nn.relu(
        jnp.dot(smri_ref[...], ws_ref[...], preferred_element_type=jnp.float32)
        + bs_ref[...])
    out = (jnp.dot(mean, wf1_ref[...], preferred_element_type=jnp.float32)
           + jnp.dot(semb, wf2_ref[...], preferred_element_type=jnp.float32)
           + bf_ref[...])
    out_ref[...] = out


def _tc_c(accp, xw2, dis, b2r, batch3, smri, Ws, bsr, Wf1, Wf2, bfr):
  nb = 10
  blk = N_NODES // nb
  return pl.pallas_call(
      _tc_c_body,
      grid=(nb,),
      in_specs=[
          pl.BlockSpec((NC, blk, NFEAT), lambda i: (0, i, 0)),
          pl.BlockSpec((blk, NFEAT), lambda i: (i, 0)),
          pl.BlockSpec((blk, 1), lambda i: (i, 0)),
          pl.BlockSpec((1, NFEAT), lambda i: (0, 0)),
          pl.BlockSpec((1, 1, blk), lambda i: (i, 0, 0)),
          pl.BlockSpec(smri.shape, lambda i: (0, 0)),
          pl.BlockSpec(Ws.shape, lambda i: (0, 0)),
          pl.BlockSpec(bsr.shape, lambda i: (0, 0)),
          pl.BlockSpec(Wf1.shape, lambda i: (0, 0)),
          pl.BlockSpec(Wf2.shape, lambda i: (0, 0)),
          pl.BlockSpec(bfr.shape, lambda i: (0, 0)),
      ],
      out_specs=pl.BlockSpec((NGRAPH, 2), lambda i: (0, 0)),
      out_shape=jax.ShapeDtypeStruct((NGRAPH, 2), jnp.float32),
      scratch_shapes=[
          pltpu.VMEM((NGRAPH, NFEAT), jnp.float32),
          pltpu.VMEM((NGRAPH, 8), jnp.float32),
      ],
  )(accp, xw2, dis, b2r, batch3, smri, Ws, bsr, Wf1, Wf2, bfr)


@jax.jit
def kernel(x, edge_index, edge_weight, batch, smri_data,
           W1, b1, W2, b2, Ws, bs, Wf, bf):
  x = x.astype(jnp.float32)
  ew = edge_weight.astype(jnp.float32)

  row2d = edge_index[0].astype(jnp.int32).reshape(NCHUNKS, CHUNK)
  col2d = edge_index[1].astype(jnp.int32).reshape(NCHUNKS, CHUNK)
  ew2d = ew.reshape(NCHUNKS, CHUNK)
  batch3 = batch.astype(jnp.int32).reshape(10, 1, N_NODES // 10)

  b1r = b1.reshape(1, NFEAT)
  b2r = b2.reshape(1, NFEAT)
  bsr = bs.reshape(1, -1)
  bfr = bf.reshape(1, -1)
  Wf1 = Wf[:NFEAT]
  Wf2 = Wf[NFEAT:]

  degp = _sc_deg(ew2d, col2d)
  xw1, xn1, dis = _tc_a(x, W1, degp)
  acc1 = _sc_msg(xn1, row2d, col2d, ew2d)
  xw2, xn2 = _tc_b(acc1, xw1, dis, b1r, W2)
  acc2 = _sc_msg(xn2, row2d, col2d, ew2d)
  out = _tc_c(acc2, xw2, dis, b2r, batch3, smri_data.astype(jnp.float32),
              Ws, bsr, Wf1, Wf2, bfr)
  return out
